# baseline (speedup 1.0000x reference)
import jax
import jax.numpy as jnp
from jax import lax
from jax.experimental import pallas as pl
from jax.experimental.pallas import tpu as pltpu
from jax.experimental.pallas import tpu_sc as plsc

_H, _W = 48, 160
_HW = _H * _W
_V = 3
_C = 64
_P = 4
_TW = 128
_ZROW = _V * _HW
_TROWS = _ZROW + 8
_NC, _NS, _L = 2, 16, 16
_NW = _NC * _NS
_B = 64
_BLK = 2 * _B
_CHUNKS = _B // _L
_ROWS = _V * _P * _B
_ISEG = 128
_NSEG = _ROWS // _ISEG


def _rsqrt(q):
    i = lax.bitcast_convert_type(q, jnp.int32)
    y = lax.bitcast_convert_type(jnp.int32(0x5F3759DF) - (i >> 1), jnp.float32)
    for _ in range(3):
        y = y * (1.5 - 0.5 * q * y * y)
    return y


def _sfa_body(table, pp, fm, out, pp_v, fm_v, idx_v, rows_v, fbuf, outb, sem):
    n_total = out.shape[1]
    per_w = n_total // _NW
    nblk = per_w // _BLK
    wid = lax.axis_index("s") * _NC + lax.axis_index("c")
    iota = lax.iota(jnp.int32, _L)

    def block(b, carry):
        blk_base = wid * per_w + b * _BLK
        for half in range(2):
            base = blk_base + half * _B
            pltpu.sync_copy(pp.at[:, pl.ds(base * _P * 2, _B * _P * 2)], pp_v)
            pltpu.sync_copy(fm.at[:, pl.ds(base * _P, _B * _P)], fm_v)

            invs = []
            ws = []
            for ch in range(_CHUNKS):
                nvec = ch * _L + iota
                for v in range(_V):
                    vf = jnp.full((_L,), v, jnp.int32)
                    cnt = jnp.zeros((_L,), jnp.int32)
                    for p in range(_P):
                        fidx = nvec * _P + p
                        m = plsc.load_gather(fm_v, [vf, fidx])
                        px = plsc.load_gather(pp_v, [vf, fidx * 2])
                        py = plsc.load_gather(pp_v, [vf, fidx * 2 + 1])
                        gi = jnp.where(m > 0, v * _HW + py * _W + px, _ZROW)
                        g = (ch * _V + v) * _P + p
                        idx_v[g // 8, pl.ds((g % 8) * _L, _L)] = gi
                        cnt = cnt + m
                    cntf = cnt.astype(jnp.float32)
                    pos = cnt > 0
                    invs.append(jnp.where(pos, 1.0 / cntf, 0.0))
                    ws.append(jnp.where(pos, 1.0, 0.0))

            copies = [
                pltpu.async_copy(
                    table.at[idx_v.at[j]], rows_v.at[pl.ds(j * _ISEG, _ISEG)], sem
                )
                for j in range(_NSEG)
            ]
            for cp in copies:
                cp.wait()

            for ch in range(_CHUNKS):
                inv = invs[ch * _V : ch * _V + _V]
                w1, w2, w3 = ws[ch * _V : ch * _V + _V]

                def chan(k, acc, ch=ch, inv=inv):
                    d12, d13, d23, n1, n2, n3 = acc
                    kf = jnp.full((_L,), k, jnp.int32)
                    fs = []
                    for v in range(_V):
                        s = None
                        for p in range(_P):
                            r0 = ((ch * _V + v) * _P + p) * _L
                            gv = plsc.load_gather(rows_v, [r0 + iota, kf])
                            s = gv if s is None else s + gv
                        fv = s * inv[v]
                        fbuf[pl.ds((v * _C + k) * _L, _L)] = fv
                        fs.append(fv)
                    f1, f2, f3 = fs
                    return (
                        d12 + f1 * f2,
                        d13 + f1 * f3,
                        d23 + f2 * f3,
                        n1 + f1 * f1,
                        n2 + f2 * f2,
                        n3 + f3 * f3,
                    )

                z = jnp.zeros((_L,), jnp.float32)
                d12, d13, d23, n1, n2, n3 = lax.fori_loop(
                    0, _C, chan, (z, z, z, z, z, z)
                )

                def cosw(d, ni, nj, wi, wj):
                    q = jnp.maximum(ni * nj, 1e-16)
                    denom = jnp.maximum(q * _rsqrt(q), 1e-8)
                    return (d / denom) * (wi * wj)

                c12 = cosw(d12, n1, n2, w1, w2)
                c13 = cosw(d13, n1, n3, w1, w3)
                c23 = cosw(d23, n2, n3, w2, w3)
                i12 = w1 * (1.0 - w2)
                i21 = w2 * (1.0 - w1)
                i13 = w1 * (1.0 - w3)
                i31 = w3 * (1.0 - w1)
                i23 = w2 * (1.0 - w3)
                i32 = w3 * (1.0 - w2)
                sixth = jnp.float32(1.0 / 6.0)
                a1 = (c12 + i12 + c13 + i13) * sixth
                a2 = (c12 + i21 + c23 + i23) * sixth
                a3 = (c13 + i31 + c23 + i32) * sixth

                def emit(k, c, half=half, ch=ch, a1=a1, a2=a2, a3=a3):
                    f1 = fbuf[pl.ds(k * _L, _L)]
                    f2 = fbuf[pl.ds((_C + k) * _L, _L)]
                    f3 = fbuf[pl.ds((2 * _C + k) * _L, _L)]
                    outb[k, pl.ds(half * _B + ch * _L, _L)] = (
                        a1 * f1 + a2 * f2 + a3 * f3
                    )
                    return c

                lax.fori_loop(0, _C, emit, 0)

        pltpu.sync_copy(outb, out.at[:, pl.ds(blk_base, _BLK)])
        return carry

    lax.fori_loop(0, nblk, block, 0)


def _sfa_call(table, pp, fm, n_total):
    mesh = plsc.VectorSubcoreMesh(
        core_axis_name="c", subcore_axis_name="s", num_cores=_NC, num_subcores=_NS
    )
    return pl.kernel(
        _sfa_body,
        out_type=jax.ShapeDtypeStruct((_C, n_total), jnp.float32),
        mesh=mesh,
        compiler_params=pltpu.CompilerParams(needs_layout_passes=False),
        scratch_types=[
            pltpu.VMEM((_V, _B * _P * 2), jnp.int32),
            pltpu.VMEM((_V, _B * _P), jnp.int32),
            pltpu.VMEM((_NSEG, _ISEG), jnp.int32),
            pltpu.VMEM((_ROWS, _TW), jnp.float32),
            pltpu.VMEM((_V * _C * _L,), jnp.float32),
            pltpu.VMEM((_C, _BLK), jnp.float32),
            pltpu.SemaphoreType.DMA,
        ],
    )(table, pp, fm)


def kernel(x2d, projected_pix, fov_mask):
    v, c, h, w = x2d.shape
    n_total = projected_pix.shape[1]
    table = jnp.transpose(x2d.reshape(v, c, h * w), (0, 2, 1)).reshape(v * h * w, c)
    table = jnp.pad(table, ((0, _TROWS - _ZROW), (0, _TW - c)))
    pp = projected_pix.astype(jnp.int32).reshape(v, n_total * _P * 2)
    fm = fov_mask.astype(jnp.int32).reshape(v, n_total * _P)
    out = _sfa_call(table, pp, fm, n_total)
    return out.reshape(c, 128, 128, 16)


# speedup vs baseline: 5.7627x; 5.7627x over previous
import jax
import jax.numpy as jnp
from jax import lax
from jax.experimental import pallas as pl
from jax.experimental.pallas import tpu as pltpu
from jax.experimental.pallas import tpu_sc as plsc

_H, _W = 48, 160
_CW = 48
_CHW = _H * _CW
_V = 3
_C = 64
_P = 4
_ZROW = _V * _CHW
_TROWS = _ZROW + 8
_NC, _NS, _L = 2, 16, 16
_NW = _NC * _NS
_B = 32
_RPB = 128 // _B
_BLK = _RPB * _B
_CHUNKS = _B // _L
_ROWS = _V * _P * _B
_ISEG = 128
_NSEG = _ROWS // _ISEG


def _rsqrt(q):
    i = lax.bitcast_convert_type(q, jnp.int32)
    y = lax.bitcast_convert_type(jnp.int32(0x5F3759DF) - (i >> 1), jnp.float32)
    for _ in range(3):
        y = y * (1.5 - 0.5 * q * y * y)
    return y


def _sfa_body(table, pp, fm, out, shared, pp_v, fm_v, idx_v, rows_v, fbuf, outb, sem):
    n_total = out.shape[1]
    per_w = n_total // _NW
    nblk = per_w // _BLK
    sid = lax.axis_index("s")
    wid = sid * _NC + lax.axis_index("c")
    iota = lax.iota(jnp.int32, _L)

    @pl.when(sid == 0)
    def _():
        pltpu.sync_copy(table, shared)

    plsc.subcore_barrier()

    def block(b, carry):
        blk_base = wid * per_w + b * _BLK
        for half in range(_RPB):
            base = blk_base + half * _B
            pltpu.sync_copy(pp.at[:, pl.ds(base * _P * 2, _B * _P * 2)], pp_v)
            pltpu.sync_copy(fm.at[:, pl.ds(base * _P, _B * _P)], fm_v)

            invs = []
            ws = []
            for ch in range(_CHUNKS):
                nvec = ch * _L + iota
                for v in range(_V):
                    vf = jnp.full((_L,), v, jnp.int32)
                    cnt = jnp.zeros((_L,), jnp.int32)
                    for p in range(_P):
                        fidx = nvec * _P + p
                        m = plsc.load_gather(fm_v, [vf, fidx])
                        px = plsc.load_gather(pp_v, [vf, fidx * 2])
                        py = plsc.load_gather(pp_v, [vf, fidx * 2 + 1])
                        gi = jnp.where(m > 0, v * _CHW + py * _CW + px, _ZROW)
                        g = (ch * _V + v) * _P + p
                        idx_v[g // 8, pl.ds((g % 8) * _L, _L)] = gi
                        cnt = cnt + m
                    cntf = cnt.astype(jnp.float32)
                    pos = cnt > 0
                    invs.append(jnp.where(pos, 1.0 / cntf, 0.0))
                    ws.append(jnp.where(pos, 1.0, 0.0))

            copies = [
                pltpu.async_copy(
                    shared.at[idx_v.at[j]], rows_v.at[pl.ds(j * _ISEG, _ISEG)], sem
                )
                for j in range(_NSEG)
            ]
            for cp in copies:
                cp.wait()

            for ch in range(_CHUNKS):
                inv = invs[ch * _V : ch * _V + _V]
                w1, w2, w3 = ws[ch * _V : ch * _V + _V]

                def chan(k, acc, ch=ch, inv=inv):
                    d12, d13, d23, n1, n2, n3 = acc
                    kf = jnp.full((_L,), k, jnp.int32)
                    fs = []
                    for v in range(_V):
                        s = None
                        for p in range(_P):
                            r0 = ((ch * _V + v) * _P + p) * _L
                            gv = plsc.load_gather(rows_v, [r0 + iota, kf])
                            s = gv if s is None else s + gv
                        fv = s * inv[v]
                        fbuf[pl.ds((v * _C + k) * _L, _L)] = fv
                        fs.append(fv)
                    f1, f2, f3 = fs
                    return (
                        d12 + f1 * f2,
                        d13 + f1 * f3,
                        d23 + f2 * f3,
                        n1 + f1 * f1,
                        n2 + f2 * f2,
                        n3 + f3 * f3,
                    )

                z = jnp.zeros((_L,), jnp.float32)
                d12, d13, d23, n1, n2, n3 = lax.fori_loop(
                    0, _C, chan, (z, z, z, z, z, z)
                )

                def cosw(d, ni, nj, wi, wj):
                    q = jnp.maximum(ni * nj, 1e-16)
                    denom = jnp.maximum(q * _rsqrt(q), 1e-8)
                    return (d / denom) * (wi * wj)

                c12 = cosw(d12, n1, n2, w1, w2)
                c13 = cosw(d13, n1, n3, w1, w3)
                c23 = cosw(d23, n2, n3, w2, w3)
                i12 = w1 * (1.0 - w2)
                i21 = w2 * (1.0 - w1)
                i13 = w1 * (1.0 - w3)
                i31 = w3 * (1.0 - w1)
                i23 = w2 * (1.0 - w3)
                i32 = w3 * (1.0 - w2)
                sixth = jnp.float32(1.0 / 6.0)
                a1 = (c12 + i12 + c13 + i13) * sixth
                a2 = (c12 + i21 + c23 + i23) * sixth
                a3 = (c13 + i31 + c23 + i32) * sixth

                def emit(k, c, half=half, ch=ch, a1=a1, a2=a2, a3=a3):
                    f1 = fbuf[pl.ds(k * _L, _L)]
                    f2 = fbuf[pl.ds((_C + k) * _L, _L)]
                    f3 = fbuf[pl.ds((2 * _C + k) * _L, _L)]
                    outb[k, pl.ds(half * _B + ch * _L, _L)] = (
                        a1 * f1 + a2 * f2 + a3 * f3
                    )
                    return c

                lax.fori_loop(0, _C, emit, 0)

        pltpu.sync_copy(outb, out.at[:, pl.ds(blk_base, _BLK)])
        return carry

    lax.fori_loop(0, nblk, block, 0)


def _sfa_call(table, pp, fm, n_total):
    mesh = plsc.VectorSubcoreMesh(
        core_axis_name="c", subcore_axis_name="s", num_cores=_NC, num_subcores=_NS
    )
    return pl.kernel(
        _sfa_body,
        out_type=jax.ShapeDtypeStruct((_C, n_total), jnp.float32),
        mesh=mesh,
        compiler_params=pltpu.CompilerParams(needs_layout_passes=False),
        scratch_types=[
            pltpu.VMEM_SHARED((_TROWS, 2 * _C), jnp.float32),
            pltpu.VMEM((_V, _B * _P * 2), jnp.int32),
            pltpu.VMEM((_V, _B * _P), jnp.int32),
            pltpu.VMEM((_NSEG, _ISEG), jnp.int32),
            pltpu.VMEM((_ROWS, 2 * _C), jnp.float32),
            pltpu.VMEM((_V * _C * _L,), jnp.float32),
            pltpu.VMEM((_C, _BLK), jnp.float32),
            pltpu.SemaphoreType.DMA,
        ],
    )(table, pp, fm)


def kernel(x2d, projected_pix, fov_mask):
    v, c, h, w = x2d.shape
    n_total = projected_pix.shape[1]
    table = jnp.transpose(
        x2d[:, :, :, :_CW].reshape(v, c, h * _CW), (0, 2, 1)
    ).reshape(v * h * _CW, c)
    table = jnp.pad(table, ((0, _TROWS - _ZROW), (0, _C)))
    pp = projected_pix.astype(jnp.int32).reshape(v, n_total * _P * 2)
    fm = fov_mask.astype(jnp.int32).reshape(v, n_total * _P)
    out = _sfa_call(table, pp, fm, n_total)
    return out.reshape(c, 128, 128, 16)


# speedup vs baseline: 8.3903x; 1.4560x over previous
import jax
import jax.numpy as jnp
from jax import lax
from jax.experimental import pallas as pl
from jax.experimental.pallas import tpu as pltpu
from jax.experimental.pallas import tpu_sc as plsc

_H, _W = 48, 160
_CW = 48
_CHW = _H * _CW
_V = 3
_C = 64
_P = 4
_ZROW = _V * _CHW
_TROWS = _ZROW + 8
_NC, _NS, _L = 2, 16, 16
_NW = _NC * _NS
_B = 16
_RPB = 128 // _B
_BLK = _RPB * _B
_CHUNKS = _B // _L
_ROWS = _V * _P * _B
_ISEG = 96
_NSEG = _ROWS // _ISEG


def _rsqrt(q):
    i = lax.bitcast_convert_type(q, jnp.int32)
    y = lax.bitcast_convert_type(jnp.int32(0x5F3759DF) - (i >> 1), jnp.float32)
    for _ in range(3):
        y = y * (1.5 - 0.5 * q * y * y)
    return y


def _sfa_body(table, pp, fm, out, shared, pp_v, fm_v, idx_v, rows_v, fbuf, outb, sem):
    n_total = out.shape[1]
    per_w = n_total // _NW
    nblk = per_w // _BLK
    sid = lax.axis_index("s")
    wid = sid * _NC + lax.axis_index("c")
    iota = lax.iota(jnp.int32, _L)

    @pl.when(sid == 0)
    def _():
        pltpu.sync_copy(table, shared)

    plsc.subcore_barrier()

    def block(b, carry):
        blk_base = wid * per_w + b * _BLK
        for half in range(_RPB):
            base = blk_base + half * _B
            pltpu.sync_copy(pp.at[:, pl.ds(base, _B)], pp_v)
            pltpu.sync_copy(fm.at[:, pl.ds(base, _B)], fm_v)

            invs = []
            ws = []
            for ch in range(_CHUNKS):
                nvec = ch * _L + iota
                for v in range(_V):
                    vf = jnp.full((_L,), v, jnp.int32)
                    cnt = jnp.zeros((_L,), jnp.int32)
                    for p in range(_P):
                        pf = jnp.full((_L,), p, jnp.int32)
                        m = plsc.load_gather(fm_v, [vf, nvec, pf])
                        px = plsc.load_gather(
                            pp_v, [vf, nvec, pf, jnp.zeros((_L,), jnp.int32)]
                        )
                        py = plsc.load_gather(
                            pp_v, [vf, nvec, pf, jnp.full((_L,), 1, jnp.int32)]
                        )
                        gi = jnp.where(m > 0, v * _CHW + py * _CW + px, _ZROW)
                        g = (ch * _V + v) * _P + p
                        idx_v[g // 6, pl.ds((g % 6) * _L, _L)] = gi
                        cnt = cnt + m
                    cntf = cnt.astype(jnp.float32)
                    pos = cnt > 0
                    invs.append(jnp.where(pos, 1.0 / cntf, 0.0))
                    ws.append(jnp.where(pos, 1.0, 0.0))

            copies = [
                pltpu.async_copy(
                    shared.at[idx_v.at[j]], rows_v.at[pl.ds(j * _ISEG, _ISEG)], sem
                )
                for j in range(_NSEG)
            ]
            for cp in copies:
                cp.wait()

            for ch in range(_CHUNKS):
                inv = invs[ch * _V : ch * _V + _V]
                w1, w2, w3 = ws[ch * _V : ch * _V + _V]

                def chan(k, acc, ch=ch, inv=inv):
                    d12, d13, d23, n1, n2, n3 = acc
                    kf = jnp.full((_L,), k, jnp.int32)
                    fs = []
                    for v in range(_V):
                        s = None
                        for p in range(_P):
                            r0 = ((ch * _V + v) * _P + p) * _L
                            gv = plsc.load_gather(rows_v, [r0 + iota, kf])
                            s = gv if s is None else s + gv
                        fv = s * inv[v]
                        fbuf[pl.ds((v * _C + k) * _L, _L)] = fv
                        fs.append(fv)
                    f1, f2, f3 = fs
                    return (
                        d12 + f1 * f2,
                        d13 + f1 * f3,
                        d23 + f2 * f3,
                        n1 + f1 * f1,
                        n2 + f2 * f2,
                        n3 + f3 * f3,
                    )

                z = jnp.zeros((_L,), jnp.float32)
                d12, d13, d23, n1, n2, n3 = lax.fori_loop(
                    0, _C, chan, (z, z, z, z, z, z)
                )

                def cosw(d, ni, nj, wi, wj):
                    q = jnp.maximum(ni * nj, 1e-16)
                    denom = jnp.maximum(q * _rsqrt(q), 1e-8)
                    return (d / denom) * (wi * wj)

                c12 = cosw(d12, n1, n2, w1, w2)
                c13 = cosw(d13, n1, n3, w1, w3)
                c23 = cosw(d23, n2, n3, w2, w3)
                i12 = w1 * (1.0 - w2)
                i21 = w2 * (1.0 - w1)
                i13 = w1 * (1.0 - w3)
                i31 = w3 * (1.0 - w1)
                i23 = w2 * (1.0 - w3)
                i32 = w3 * (1.0 - w2)
                sixth = jnp.float32(1.0 / 6.0)
                a1 = (c12 + i12 + c13 + i13) * sixth
                a2 = (c12 + i21 + c23 + i23) * sixth
                a3 = (c13 + i31 + c23 + i32) * sixth

                def emit(k, c, half=half, ch=ch, a1=a1, a2=a2, a3=a3):
                    f1 = fbuf[pl.ds(k * _L, _L)]
                    f2 = fbuf[pl.ds((_C + k) * _L, _L)]
                    f3 = fbuf[pl.ds((2 * _C + k) * _L, _L)]
                    outb[k, pl.ds(half * _B + ch * _L, _L)] = (
                        a1 * f1 + a2 * f2 + a3 * f3
                    )
                    return c

                lax.fori_loop(0, _C, emit, 0)

        pltpu.sync_copy(outb, out.at[:, pl.ds(blk_base, _BLK)])
        return carry

    lax.fori_loop(0, nblk, block, 0)


def _sfa_call(table, pp, fm, n_total):
    mesh = plsc.VectorSubcoreMesh(
        core_axis_name="c", subcore_axis_name="s", num_cores=_NC, num_subcores=_NS
    )
    return pl.kernel(
        _sfa_body,
        out_type=jax.ShapeDtypeStruct((_C, n_total), jnp.float32),
        mesh=mesh,
        compiler_params=pltpu.CompilerParams(needs_layout_passes=False),
        scratch_types=[
            pltpu.VMEM_SHARED((_TROWS, 2 * _C), jnp.float32),
            pltpu.VMEM((_V, _B, _P, 2), jnp.int32),
            pltpu.VMEM((_V, _B, _P), jnp.int32),
            pltpu.VMEM((_NSEG, _ISEG), jnp.int32),
            pltpu.VMEM((_ROWS, 2 * _C), jnp.float32),
            pltpu.VMEM((_V * _C * _L,), jnp.float32),
            pltpu.VMEM((_C, _BLK), jnp.float32),
            pltpu.SemaphoreType.DMA,
        ],
    )(table, pp, fm)


def kernel(x2d, projected_pix, fov_mask):
    v, c, h, w = x2d.shape
    n_total = projected_pix.shape[1]
    table = jnp.transpose(
        x2d[:, :, :, :_CW].reshape(v, c, h * _CW), (0, 2, 1)
    ).reshape(v * h * _CW, c)
    table = jnp.pad(table, ((0, _TROWS - _ZROW), (0, _C)))
    pp = projected_pix.astype(jnp.int32)
    fm = fov_mask.astype(jnp.int32)
    out = _sfa_call(table, pp, fm, n_total)
    return out.reshape(c, 128, 128, 16)
